# initial kernel scaffold (unmeasured)
import jax
import jax.numpy as jnp
from jax import lax
from jax.experimental import pallas as pl
from jax.experimental.pallas import tpu as pltpu

N_DEV = 4


def kernel(x, w_mat, scale_x, scale_w):
    m_total, k_shard = x.shape
    k_total, n_total = w_mat.shape
    m_per = m_total // N_DEV

    def body(x_ref, w_ref, sx_ref, sw_ref, out_ref, comm_ref, send_sems, recv_sems):
        my = lax.axis_index("i")

        barrier_sem = pltpu.get_barrier_semaphore()
        for d in range(1, N_DEV):
            pl.semaphore_signal(
                barrier_sem, inc=1,
                device_id=((my + d) % N_DEV,),
                device_id_type=pl.DeviceIdType.MESH,
            )
        pl.semaphore_wait(barrier_sem, N_DEV - 1)

        rdmas = []
        for d in range(1, N_DEV):
            tgt = (my + d) % N_DEV
            rdma = pltpu.make_async_remote_copy(
                src_ref=x_ref.at[pl.ds(tgt * m_per, m_per)],
                dst_ref=comm_ref.at[d - 1],
                send_sem=send_sems.at[d - 1],
                recv_sem=recv_sems.at[d - 1],
                device_id=(tgt,),
                device_id_type=pl.DeviceIdType.MESH,
            )
            rdma.start()
            rdmas.append(rdma)

        scale = sx_ref[0] * sw_ref[0]

        acc = lax.dot_general(
            x_ref[pl.ds(my * m_per, m_per)],
            w_ref[pl.ds(my * k_shard, k_shard)],
            (((1,), (0,)), ((), ())),
            preferred_element_type=jnp.int32,
        )

        for d in range(1, N_DEV):
            rdmas[d - 1].wait()
            src = (my - d) % N_DEV
            acc = acc + lax.dot_general(
                comm_ref[d - 1],
                w_ref[pl.ds(src * k_shard, k_shard)],
                (((1,), (0,)), ((), ())),
                preferred_element_type=jnp.int32,
            )

        out_ref[...] = jnp.maximum(acc.astype(jnp.float32) * scale, 0.0)

    return pl.pallas_call(
        body,
        out_shape=jax.ShapeDtypeStruct((m_per, n_total), jnp.float32),
        in_specs=[
            pl.BlockSpec(memory_space=pltpu.VMEM),
            pl.BlockSpec(memory_space=pltpu.VMEM),
            pl.BlockSpec(memory_space=pltpu.SMEM),
            pl.BlockSpec(memory_space=pltpu.SMEM),
        ],
        out_specs=pl.BlockSpec(memory_space=pltpu.VMEM),
        scratch_shapes=[
            pltpu.VMEM((N_DEV - 1, m_per, k_shard), jnp.int8),
            pltpu.SemaphoreType.DMA((N_DEV - 1,)),
            pltpu.SemaphoreType.DMA((N_DEV - 1,)),
        ],
        compiler_params=pltpu.CompilerParams(collective_id=0),
    )(x, w_mat, scale_x, scale_w)


# baseline (device time: 145982 ns/iter reference)
import jax
import jax.numpy as jnp
from jax import lax
from jax.experimental import pallas as pl
from jax.experimental.pallas import tpu as pltpu

N_DEV = 4
N_BLK = 512

_SLOT_OF_D = {1: 1, 3: 2, 2: 3}


def kernel(x, w_mat, scale_x, scale_w):
    m_total, k_shard = x.shape
    k_total, n_total = w_mat.shape
    m_per = m_total // N_DEV
    nt_count = n_total // N_BLK
    num_tiles = N_DEV * nt_count

    def body(x_ref, w_hbm, sx_ref, sw_ref, out_ref,
             chunk_buf, wbuf, send_sems, recv_sems, wsems):
        my = lax.axis_index("i")

        barrier_sem = pltpu.get_barrier_semaphore()
        for d in range(1, N_DEV):
            pl.semaphore_signal(
                barrier_sem, inc=1,
                device_id=((my + d) % N_DEV,),
                device_id_type=pl.DeviceIdType.MESH,
            )
        pl.semaphore_wait(barrier_sem, N_DEV - 1)

        def a2a_rdma(d):
            tgt = (my + d) % N_DEV
            return pltpu.make_async_remote_copy(
                src_ref=x_ref.at[pl.ds(tgt * m_per, m_per)],
                dst_ref=chunk_buf.at[_SLOT_OF_D[d]],
                send_sem=send_sems.at[_SLOT_OF_D[d] - 1],
                recv_sem=recv_sems.at[_SLOT_OF_D[d] - 1],
                device_id=(tgt,),
                device_id_type=pl.DeviceIdType.MESH,
            )

        a2a_rdma(1).start()
        a2a_rdma(3).start()

        chunk_buf[0] = x_ref[pl.ds(my * m_per, m_per)]

        scale = sx_ref[0] * sw_ref[0]

        def src_k(si):
            gray = si ^ (si >> 1)
            return (my - gray) % N_DEV

        def w_dma(j, slot):
            return pltpu.make_async_copy(
                w_hbm.at[pl.ds(src_k(j // nt_count) * k_shard, k_shard),
                         pl.ds((j % nt_count) * N_BLK, N_BLK)],
                wbuf.at[slot],
                wsems.at[slot],
            )

        w_dma(0, 0).start()

        def tile_step(i, carry):
            si = i // nt_count
            nt = i % nt_count

            @pl.when(i + 1 < num_tiles)
            def _():
                w_dma(i + 1, (i + 1) % 2).start()

            @pl.when(i == nt_count)
            def _():
                a2a_rdma(1).wait_send()
                a2a_rdma(3).wait_send()
                a2a_rdma(2).start()
                a2a_rdma(1).wait_recv()

            @pl.when(i == 2 * nt_count)
            def _():
                a2a_rdma(3).wait_recv()

            @pl.when(i == 3 * nt_count)
            def _():
                a2a_rdma(2).wait_recv()

            w_dma(i, i % 2).wait()

            dot = lax.dot_general(
                chunk_buf[si], wbuf[i % 2],
                (((1,), (0,)), ((), ())),
                preferred_element_type=jnp.int32,
            )
            contrib = dot.astype(jnp.float32) * scale
            nds = pl.ds(nt * N_BLK, N_BLK)
            acc = jnp.where(si == 0, contrib, out_ref[:, nds] + contrib)
            out_ref[:, nds] = jnp.where(si == N_DEV - 1,
                                        jnp.maximum(acc, 0.0), acc)
            return carry

        lax.fori_loop(0, num_tiles, tile_step, 0)

        a2a_rdma(2).wait_send()

    return pl.pallas_call(
        body,
        out_shape=jax.ShapeDtypeStruct((m_per, n_total), jnp.float32),
        in_specs=[
            pl.BlockSpec(memory_space=pltpu.VMEM),
            pl.BlockSpec(memory_space=pl.ANY),
            pl.BlockSpec(memory_space=pltpu.SMEM),
            pl.BlockSpec(memory_space=pltpu.SMEM),
        ],
        out_specs=pl.BlockSpec(memory_space=pltpu.VMEM),
        scratch_shapes=[
            pltpu.VMEM((N_DEV, m_per, k_shard), jnp.int8),
            pltpu.VMEM((2, k_shard, N_BLK), jnp.int8),
            pltpu.SemaphoreType.DMA((N_DEV - 1,)),
            pltpu.SemaphoreType.DMA((N_DEV - 1,)),
            pltpu.SemaphoreType.DMA((2,)),
        ],
        compiler_params=pltpu.CompilerParams(
            collective_id=0,
            vmem_limit_bytes=56 * 1024 * 1024,
        ),
    )(x, w_mat, scale_x, scale_w)
